# baseline (speedup 1.0000x reference)
import functools

import jax
import jax.numpy as jnp
from jax import lax
from jax.experimental import pallas as pl
from jax.experimental.pallas import tpu as pltpu
from jax.experimental.pallas import tpu_sc as plsc


def _encode(x, w, b2d, block_rows=1000):
  m, d = x.shape
  h = w.shape[1]

  def body(x_ref, w_ref, b_ref, o_ref):
    acc = jnp.dot(x_ref[...], w_ref[...], preferred_element_type=jnp.float32)
    o_ref[...] = jnp.maximum(acc + b_ref[...], 0.0)

  return pl.pallas_call(
      body,
      grid=(m // block_rows,),
      in_specs=[
          pl.BlockSpec((block_rows, d), lambda i: (i, 0)),
          pl.BlockSpec((d, h), lambda i: (0, 0)),
          pl.BlockSpec((1, h), lambda i: (0, 0)),
      ],
      out_specs=pl.BlockSpec((block_rows, h), lambda i: (i, 0)),
      out_shape=jax.ShapeDtypeStruct((m, h), jnp.float32),
  )(x, w, b2d)


def _combine(partial, n, block_rows=1000):
  _, _, d = partial.shape

  def body(p_ref, o_ref):
    o_ref[...] = p_ref[0] + p_ref[1]

  return pl.pallas_call(
      body,
      grid=(n // block_rows,),
      in_specs=[pl.BlockSpec((2, block_rows, d), lambda i: (0, i, 0))],
      out_specs=pl.BlockSpec((block_rows, d), lambda i: (i, 0)),
      out_shape=jax.ShapeDtypeStruct((n, d), jnp.float32),
  )(partial)


def _edge_aggregate(embed, src, dst, zeros_tile, n_pad):
  n, d = embed.shape
  e = src.shape[0]
  num_cores = 2
  num_subcores = 16
  nw = num_cores * num_subcores
  chunk = 128
  nq = 4
  edges_per_tile = -(-e // (nw * nq * chunk)) * nq * chunk
  e_pad = edges_per_tile * nw
  cpq = edges_per_tile // (nq * chunk)
  if e_pad > e:
    src = jnp.concatenate([src, jnp.zeros((e_pad - e,), jnp.int32)])
    dst = jnp.concatenate([dst, jnp.full((e_pad - e,), n, jnp.int32)])
  src4 = src.reshape(nw, nq, cpq, chunk)
  dst4 = dst.reshape(nw, nq, cpq, chunk)
  rows_per_tile = n_pad // num_subcores

  mesh = plsc.VectorSubcoreMesh(core_axis_name="c", subcore_axis_name="s")

  @functools.partial(
      pl.kernel,
      mesh=mesh,
      out_type=jax.ShapeDtypeStruct((num_cores, n_pad, d), jnp.float32),
      scratch_types=[
          pltpu.VMEM_SHARED((n_pad, d), jnp.float32),
          pltpu.VMEM((cpq, chunk), jnp.int32),
          pltpu.VMEM((cpq, chunk), jnp.int32),
          pltpu.VMEM((cpq, chunk), jnp.int32),
          pltpu.VMEM((cpq, chunk), jnp.int32),
          pltpu.VMEM((chunk, d), jnp.float32),
          pltpu.VMEM((chunk, d), jnp.float32),
          pltpu.SemaphoreType.DMA,
          pltpu.SemaphoreType.DMA,
          pltpu.SemaphoreType.DMA,
          pltpu.SemaphoreType.DMA,
      ],
  )
  def agg(embed_hbm, src_hbm, dst_hbm, zero_hbm, out_hbm,
          acc, si0, di0, si1, di1, rows0, rows1, sem0, sem1, isem0, isem1):
    c = lax.axis_index("c")
    s = lax.axis_index("s")
    wid = c * num_subcores + s

    idx_sets = ((si0, di0, isem0), (si1, di1, isem1))
    row_bufs = ((rows0, sem0), (rows1, sem1))

    def idx_load(q, sets):
      si, di, isem = sets
      pltpu.async_copy(src_hbm.at[wid, q], si, isem)
      pltpu.async_copy(dst_hbm.at[wid, q], di, isem)

    def idx_wait(q, sets):
      si, di, isem = sets
      pltpu.make_async_copy(src_hbm.at[wid, q], si, isem).wait()
      pltpu.make_async_copy(dst_hbm.at[wid, q], di, isem).wait()

    idx_load(0, idx_sets[0])
    idx_load(1, idx_sets[1])
    pltpu.sync_copy(zero_hbm, acc.at[pl.ds(s * rows_per_tile, rows_per_tile)])
    plsc.subcore_barrier()

    for q in range(nq):
      si, di, isem = idx_sets[q % 2]
      idx_wait(q, idx_sets[q % 2])

      def start_gather(g, rows, sem):
        pltpu.async_copy(embed_hbm.at[si.at[g]], rows, sem)

      def wait_gather(g, rows, sem):
        pltpu.make_async_copy(embed_hbm.at[si.at[g]], rows, sem).wait()

      start_gather(0, rows0, sem0)
      start_gather(1, rows1, sem1)

      @pl.loop(0, cpq - 2, step=2)
      def _(g):
        for b, (rows, sem) in enumerate(row_bufs):
          wait_gather(g + b, rows, sem)
          pltpu.sync_copy(rows, acc.at[di.at[g + b]], add=True)
          start_gather(g + 2 + b, rows, sem)

      for b, (rows, sem) in enumerate(row_bufs):
        g = cpq - 2 + b
        wait_gather(g, rows, sem)
        pltpu.sync_copy(rows, acc.at[di.at[g]], add=True)

      if q + 2 < nq:
        idx_load(q + 2, idx_sets[q % 2])

    plsc.subcore_barrier()
    sl = pl.ds(s * rows_per_tile, rows_per_tile)
    pltpu.sync_copy(acc.at[sl], out_hbm.at[c, sl])

  return agg(embed, src4, dst4, zeros_tile)


def kernel(state, internal, edge_index, W_enc, b_enc):
  n, d = state.shape
  n_pad = ((n + 127) // 128) * 128
  b2d = b_enc.reshape(1, -1)
  src = edge_index[0]
  dst = edge_index[1]
  zeros_tile = jnp.zeros((n_pad // 16, d), jnp.float32)

  state_embed = _encode(state, W_enc, b2d)
  partial = _edge_aggregate(state_embed, src, dst, zeros_tile, n_pad)
  internal_embed = _encode(internal, W_enc, b2d)
  neigh_sum = _combine(partial, n)
  return (state_embed, internal_embed, neigh_sum)


# speedup vs baseline: 3.0066x; 3.0066x over previous
import functools

import jax
import jax.numpy as jnp
from jax import lax
from jax.experimental import pallas as pl
from jax.experimental.pallas import tpu as pltpu
from jax.experimental.pallas import tpu_sc as plsc


def _encode(x, w, b2d, block_rows=1000):
  m, d = x.shape
  h = w.shape[1]

  def body(x_ref, w_ref, b_ref, o_ref):
    acc = jnp.dot(x_ref[...], w_ref[...], preferred_element_type=jnp.float32)
    o_ref[...] = jnp.maximum(acc + b_ref[...], 0.0)

  return pl.pallas_call(
      body,
      grid=(m // block_rows,),
      in_specs=[
          pl.BlockSpec((block_rows, d), lambda i: (i, 0)),
          pl.BlockSpec((d, h), lambda i: (0, 0)),
          pl.BlockSpec((1, h), lambda i: (0, 0)),
      ],
      out_specs=pl.BlockSpec((block_rows, h), lambda i: (i, 0)),
      out_shape=jax.ShapeDtypeStruct((m, h), jnp.float32),
  )(x, w, b2d)


def _combine(partial, n, block_rows=1000):
  _, _, d = partial.shape

  def body(p_ref, o_ref):
    o_ref[...] = p_ref[0] + p_ref[1]

  return pl.pallas_call(
      body,
      grid=(n // block_rows,),
      in_specs=[pl.BlockSpec((2, block_rows, d), lambda i: (0, i, 0))],
      out_specs=pl.BlockSpec((block_rows, d), lambda i: (i, 0)),
      out_shape=jax.ShapeDtypeStruct((n, d), jnp.float32),
  )(partial)


def _edge_aggregate(embed, src, dst, zeros_tile, n_pad):
  n, d = embed.shape
  e = src.shape[0]
  num_cores = 2
  num_subcores = 16
  nw = num_cores * num_subcores
  chunk = 128
  nq = 4
  edges_per_tile = -(-e // (nw * nq * chunk)) * nq * chunk
  e_pad = edges_per_tile * nw
  cpq = edges_per_tile // (nq * chunk)
  if e_pad > e:
    pad_i = jnp.arange(e_pad - e, dtype=jnp.int32)
    src = jnp.concatenate([src, pad_i % n])
    dst = jnp.concatenate([dst, n + pad_i % (n_pad - n)])
  src4 = src.reshape(nw, nq, cpq, chunk)
  dst4 = dst.reshape(nw, nq, cpq, chunk)
  rows_per_tile = n_pad // num_subcores

  mesh = plsc.VectorSubcoreMesh(core_axis_name="c", subcore_axis_name="s")

  @functools.partial(
      pl.kernel,
      mesh=mesh,
      out_type=jax.ShapeDtypeStruct((num_cores, n_pad, d), jnp.float32),
      scratch_types=[
          pltpu.VMEM_SHARED((n_pad, d), jnp.float32),
          pltpu.VMEM((cpq, chunk), jnp.int32),
          pltpu.VMEM((cpq, chunk), jnp.int32),
          pltpu.VMEM((cpq, chunk), jnp.int32),
          pltpu.VMEM((cpq, chunk), jnp.int32),
          pltpu.VMEM((chunk, d), jnp.float32),
          pltpu.VMEM((chunk, d), jnp.float32),
          pltpu.SemaphoreType.DMA,
          pltpu.SemaphoreType.DMA,
          pltpu.SemaphoreType.DMA,
          pltpu.SemaphoreType.DMA,
      ],
  )
  def agg(embed_hbm, src_hbm, dst_hbm, zero_hbm, out_hbm,
          acc, si0, di0, si1, di1, rows0, rows1, sem0, sem1, isem0, isem1):
    c = lax.axis_index("c")
    s = lax.axis_index("s")
    wid = c * num_subcores + s

    idx_sets = ((si0, di0, isem0), (si1, di1, isem1))
    row_bufs = ((rows0, sem0), (rows1, sem1))

    def idx_load(q, sets):
      si, di, isem = sets
      pltpu.async_copy(src_hbm.at[wid, q], si, isem)
      pltpu.async_copy(dst_hbm.at[wid, q], di, isem)

    def idx_wait(q, sets):
      si, di, isem = sets
      pltpu.make_async_copy(src_hbm.at[wid, q], si, isem).wait()
      pltpu.make_async_copy(dst_hbm.at[wid, q], di, isem).wait()

    idx_load(0, idx_sets[0])
    idx_load(1, idx_sets[1])
    pltpu.sync_copy(zero_hbm, acc.at[pl.ds(s * rows_per_tile, rows_per_tile)])
    plsc.subcore_barrier()

    for q in range(nq):
      si, di, isem = idx_sets[q % 2]
      idx_wait(q, idx_sets[q % 2])

      def start_gather(g, rows, sem):
        pltpu.async_copy(embed_hbm.at[si.at[g]], rows, sem)

      def wait_gather(g, rows, sem):
        pltpu.make_async_copy(embed_hbm.at[si.at[g]], rows, sem).wait()

      start_gather(0, rows0, sem0)
      start_gather(1, rows1, sem1)

      @pl.loop(0, cpq - 2, step=2)
      def _(g):
        for b, (rows, sem) in enumerate(row_bufs):
          wait_gather(g + b, rows, sem)
          pltpu.sync_copy(rows, acc.at[di.at[g + b]], add=True)
          start_gather(g + 2 + b, rows, sem)

      for b, (rows, sem) in enumerate(row_bufs):
        g = cpq - 2 + b
        wait_gather(g, rows, sem)
        pltpu.sync_copy(rows, acc.at[di.at[g]], add=True)

      if q + 2 < nq:
        idx_load(q + 2, idx_sets[q % 2])

    plsc.subcore_barrier()
    sl = pl.ds(s * rows_per_tile, rows_per_tile)
    pltpu.sync_copy(acc.at[sl], out_hbm.at[c, sl])

  return agg(embed, src4, dst4, zeros_tile)


def kernel(state, internal, edge_index, W_enc, b_enc):
  n, d = state.shape
  n_pad = ((n + 127) // 128) * 128
  b2d = b_enc.reshape(1, -1)
  src = edge_index[0]
  dst = edge_index[1]
  zeros_tile = jnp.zeros((n_pad // 16, d), jnp.float32)

  state_embed = _encode(state, W_enc, b2d)
  partial = _edge_aggregate(state_embed, src, dst, zeros_tile, n_pad)
  internal_embed = _encode(internal, W_enc, b2d)
  neigh_sum = _combine(partial, n)
  return (state_embed, internal_embed, neigh_sum)
